# baseline (speedup 1.0000x reference)
import jax
import jax.numpy as jnp
from jax import lax
from jax.experimental import pallas as pl
from jax.experimental.pallas import tpu as pltpu
from jax.experimental.pallas import tpu_sc as plsc

N_NODES = 100000
N_EDGES = 6400000
NC, NS, LANES = 2, 16, 16
NW = NC * NS
ROW = 128
CH_ROWS = 8
CH = CH_ROWS * ROW
ROWS = N_EDGES // ROW
CHUNKS = 195
TILE_ROWS = CHUNKS * CH_ROWS
EXTRA_ROW0 = NW * TILE_ROWS
N_EXTRA = (ROWS - EXTRA_ROW0) // CH_ROWS
SEG = 6256
ACC_N = SEG * NS
TAB = 128
NBUF = 3
OBUF = 2
PIECE = 2048
PIECES = ((0, 2048), (2048, 2048), (4096, 2048), (6144, 112))


def _zbl_body(z_hbm, ei_hbm, dist_hbm, ptab_hbm, par_hbm,
              out_hbm,
              pk_v, ptab_v, par_v, s_v, r_v, d_v, o_v, zero_v, nbuf_v,
              acc_sh, in_sems, sc_sems):
    cid = lax.axis_index("c")
    sid = lax.axis_index("s")
    wid = sid * NC + cid
    row0 = wid * TILE_ROWS

    pltpu.sync_copy(ptab_hbm, ptab_v)
    pltpu.sync_copy(par_hbm, par_v)

    def issue_inputs(c, b):
        rb = row0 + c * CH_ROWS
        pltpu.async_copy(ei_hbm.at[pl.ds(rb, CH_ROWS)], s_v.at[b],
                         in_sems.at[b])
        pltpu.async_copy(ei_hbm.at[pl.ds(ROWS + rb, CH_ROWS)], r_v.at[b],
                         in_sems.at[b])
        pltpu.async_copy(dist_hbm.at[pl.ds(rb, CH_ROWS)], d_v.at[b],
                         in_sems.at[b])

    def wait_inputs(b):
        pltpu.make_async_copy(ei_hbm.at[pl.ds(0, CH_ROWS)], s_v.at[b],
                              in_sems.at[b]).wait()
        pltpu.make_async_copy(ei_hbm.at[pl.ds(0, CH_ROWS)], r_v.at[b],
                              in_sems.at[b]).wait()
        pltpu.make_async_copy(dist_hbm.at[pl.ds(0, CH_ROWS)], d_v.at[b],
                              in_sems.at[b]).wait()

    def issue_scatter(ib, ob):
        for j in range(CH_ROWS):
            pltpu.async_copy(o_v.at[ob, j], acc_sh.at[r_v.at[ib, j]],
                             sc_sems.at[ob], add=True)

    def wait_scatter(ib, ob):
        for j in range(CH_ROWS):
            pltpu.make_async_copy(o_v.at[ob, j], acc_sh.at[r_v.at[ib, j]],
                                  sc_sems.at[ob]).wait()

    issue_inputs(0, 0)
    issue_inputs(1, 1)

    off = sid * SEG
    for po, pn in PIECES:
        pltpu.sync_copy(z_hbm.at[pl.ds(off + po, pn)],
                        nbuf_v.at[pl.ds(0, pn)])

        def _pk(i, c):
            zq = nbuf_v[pl.ds(i * LANES, LANES)]
            zero_v[pl.ds(i * LANES, LANES)] = plsc.bitcast(
                plsc.load_gather(ptab_v, [zq]), jnp.float32)
            return c
        lax.fori_loop(0, pn // LANES, _pk, 0, unroll=2)
        pltpu.sync_copy(zero_v.at[pl.ds(0, pn)],
                        out_hbm.at[pl.ds(cid * ACC_N + off + po, pn)])

    def _zero(i, c):
        zero_v[pl.ds(i * LANES, LANES)] = jnp.zeros((LANES,), jnp.float32)
        return c
    lax.fori_loop(0, PIECE // LANES, _zero, 0)
    for po, pn in PIECES:
        pltpu.sync_copy(zero_v.at[pl.ds(0, pn)],
                        acc_sh.at[pl.ds(off + po, pn)])
    plsc.subcore_barrier()
    pltpu.sync_copy(out_hbm.at[pl.ds(cid * ACC_N, ACC_N)], pk_v)

    s_x = par_v[0]
    c0, c1, c2, c3 = par_v[1], par_v[2], par_v[3], par_v[4]
    nd0, nd1, nd2, nd3 = par_v[5], par_v[6], par_v[7], par_v[8]
    inv_rsc = par_v[9]

    def compute_chunk(ib, ob):
        sref, rref, dref, oref = s_v.at[ib], r_v.at[ib], d_v.at[ib], o_v.at[ob]

        def _vec(i, c):
            row = i // (ROW // LANES)
            col = (i % (ROW // LANES)) * LANES
            s = sref[row, pl.ds(col, LANES)]
            r = rref[row, pl.ds(col, LANES)]
            dd = dref[row, pl.ds(col, LANES)]
            pku = plsc.bitcast(plsc.load_gather(pk_v, [s]), jnp.int32)
            pkv = plsc.bitcast(plsc.load_gather(pk_v, [r]), jnp.int32)
            zz = ((pku & 127) * (pkv & 127)).astype(jnp.float32)
            rq = ((lax.shift_right_logical(pku, 7) & 0x1FFF)
                  + (lax.shift_right_logical(pkv, 7) & 0x1FFF))
            pq = (lax.shift_right_logical(pku, 20)
                  + lax.shift_right_logical(pkv, 20))
            x = dd * pq.astype(jnp.float32) * s_x
            phi = (c0 * jnp.exp(nd0 * x) + c1 * jnp.exp(nd1 * x)
                   + c2 * jnp.exp(nd2 * x) + c3 * jnp.exp(nd3 * x))
            y = dd * inv_rsc / rq.astype(jnp.float32)
            y2 = y * y
            y4 = y2 * y2
            y6 = y4 * y2
            env = 1.0 - 28.0 * y6 + 48.0 * y6 * y - 21.0 * y4 * y4
            env = jnp.where(y < 1.0, env, 0.0)
            oref[row, pl.ds(col, LANES)] = 7.1998 * zz * phi * env / dd
            return c
        lax.fori_loop(0, CH // LANES, _vec, 0, unroll=2)

    def _group(g, carry):
        for b in range(6):
            c = g * 6 + b
            ib, ob = b % NBUF, b % OBUF
            wait_inputs(ib)
            compute_chunk(ib, ob)
            issue_scatter(ib, ob)
            @pl.when(c >= 1)
            def _():
                wait_scatter((b - 1) % NBUF, (b - 1) % OBUF)

            issue_inputs(c + 2, (b + 2) % NBUF)
        return carry

    lax.fori_loop(0, (CHUNKS // 6) * 6 // 6, _group, 0)
    for c in range((CHUNKS // 6) * 6, CHUNKS):
        ib, ob = c % NBUF, c % OBUF
        wait_inputs(ib)
        compute_chunk(ib, ob)
        issue_scatter(ib, ob)
        wait_scatter((c - 1) % NBUF, (c - 1) % OBUF)
        if c + 2 < CHUNKS:
            issue_inputs(c + 2, (c + 2) % NBUF)
    wait_scatter((CHUNKS - 1) % NBUF, (CHUNKS - 1) % OBUF)

    @pl.when(wid < N_EXTRA)
    def _():
        rbx = EXTRA_ROW0 + wid * CH_ROWS
        pltpu.sync_copy(ei_hbm.at[pl.ds(rbx, CH_ROWS)], s_v.at[0])
        pltpu.sync_copy(ei_hbm.at[pl.ds(ROWS + rbx, CH_ROWS)], r_v.at[0])
        pltpu.sync_copy(dist_hbm.at[pl.ds(rbx, CH_ROWS)], d_v.at[0])
        compute_chunk(0, 0)
        issue_scatter(0, 0)
        wait_scatter(0, 0)

    plsc.subcore_barrier()
    for po, pn in PIECES:
        pltpu.sync_copy(acc_sh.at[pl.ds(off + po, pn)],
                        zero_v.at[pl.ds(0, pn)])
        pltpu.sync_copy(zero_v.at[pl.ds(0, pn)],
                        out_hbm.at[pl.ds(cid * ACC_N + off + po, pn)])


def kernel(z, edge_distance, edge_index, a_factor, Z_power, screen_coefs,
           screen_exps, covalent_radii):
    idx = jnp.arange(TAB, dtype=jnp.float32)
    zpow = idx ** Z_power
    rad = jnp.pad(covalent_radii.astype(jnp.float32),
                  (0, TAB - covalent_radii.shape[0]))
    p_scale = jnp.max(zpow) / 4095.0
    r_scale = jnp.max(rad) / 8191.0
    pq = jnp.round(zpow / p_scale).astype(jnp.uint32)
    rq = jnp.round(rad / r_scale).astype(jnp.uint32)
    ptab = (pq << 20) | (rq << 7) | jnp.arange(TAB, dtype=jnp.uint32)
    ptab = lax.bitcast_convert_type(ptab, jnp.int32)

    inv_ab = 1.0 / (0.529 * a_factor.astype(jnp.float32))
    par = jnp.concatenate([
        (p_scale * inv_ab)[None],
        screen_coefs.astype(jnp.float32),
        -screen_exps.astype(jnp.float32),
        (1.0 / r_scale)[None],
    ])
    par2 = jnp.broadcast_to(par[:, None], (10, LANES)).astype(jnp.float32)

    z_pad = jnp.pad(z.astype(jnp.int32), (0, ACC_N - N_NODES))
    ei = edge_index.reshape(2 * ROWS, ROW)
    dist = edge_distance.astype(jnp.float32).reshape(ROWS, ROW)

    mesh = plsc.VectorSubcoreMesh(core_axis_name="c", subcore_axis_name="s",
                                  num_cores=NC, num_subcores=NS)
    run = pl.kernel(
        _zbl_body,
        out_type=jax.ShapeDtypeStruct((NC * ACC_N,), jnp.float32),
        mesh=mesh,
        compiler_params=pltpu.CompilerParams(needs_layout_passes=False),
        scratch_types=[
            pltpu.VMEM((ACC_N,), jnp.float32),
            pltpu.VMEM((TAB,), jnp.int32),
            pltpu.VMEM((10, LANES), jnp.float32),
            pltpu.VMEM((NBUF, CH_ROWS, ROW), jnp.int32),
            pltpu.VMEM((NBUF, CH_ROWS, ROW), jnp.int32),
            pltpu.VMEM((NBUF, CH_ROWS, ROW), jnp.float32),
            pltpu.VMEM((OBUF, CH_ROWS, ROW), jnp.float32),
            pltpu.VMEM((PIECE,), jnp.float32),
            pltpu.VMEM((PIECE,), jnp.int32),
            pltpu.VMEM_SHARED((ACC_N,), jnp.float32),
            pltpu.SemaphoreType.DMA((NBUF,)),
            pltpu.SemaphoreType.DMA((OBUF,)),
        ],
    )
    partial = run(z_pad, ei, dist, ptab, par2)
    return partial[:N_NODES] + partial[ACC_N:ACC_N + N_NODES]


# speedup vs baseline: 1.1031x; 1.1031x over previous
import jax
import jax.numpy as jnp
from jax import lax
from jax.experimental import pallas as pl
from jax.experimental.pallas import tpu as pltpu
from jax.experimental.pallas import tpu_sc as plsc

N_NODES = 100000
N_EDGES = 6400000
NC, NS, LANES = 2, 16, 16
NW = NC * NS
ROW = 128
CH_ROWS = 8
CH = CH_ROWS * ROW
ROWS = N_EDGES // ROW
CHUNKS = 195
TILE_ROWS = CHUNKS * CH_ROWS
EXTRA_ROW0 = NW * TILE_ROWS
N_EXTRA = (ROWS - EXTRA_ROW0) // CH_ROWS
SEG = 6256
ACC_N = SEG * NS
TAB = 128
NBUF = 4
OBUF = 3
PERIOD = 12
PIECE = 2048
PIECES = ((0, 2048), (2048, 2048), (4096, 2048), (6144, 112))


def _zbl_body(z_hbm, send_hbm, recv_hbm, dist_hbm, ptab_hbm, par_hbm,
              out_hbm,
              pk_v, ptab_v, par_v, s_v, r_v, d_v, o_v, zero_v, nbuf_v,
              acc_sh, in_sems, sc_sems):
    cid = lax.axis_index("c")
    sid = lax.axis_index("s")
    wid = sid * NC + cid
    row0 = wid * TILE_ROWS

    pltpu.sync_copy(ptab_hbm, ptab_v)
    pltpu.sync_copy(par_hbm, par_v)

    def issue_inputs(c, b):
        rb = row0 + c * CH_ROWS
        pltpu.async_copy(send_hbm.at[pl.ds(rb, CH_ROWS)], s_v.at[b],
                         in_sems.at[b])
        pltpu.async_copy(recv_hbm.at[pl.ds(rb, CH_ROWS)], r_v.at[b],
                         in_sems.at[b])
        pltpu.async_copy(dist_hbm.at[pl.ds(rb, CH_ROWS)], d_v.at[b],
                         in_sems.at[b])

    def wait_inputs(b):
        pltpu.make_async_copy(send_hbm.at[pl.ds(0, CH_ROWS)], s_v.at[b],
                              in_sems.at[b]).wait()
        pltpu.make_async_copy(recv_hbm.at[pl.ds(0, CH_ROWS)], r_v.at[b],
                              in_sems.at[b]).wait()
        pltpu.make_async_copy(dist_hbm.at[pl.ds(0, CH_ROWS)], d_v.at[b],
                              in_sems.at[b]).wait()

    def issue_scatter(ib, ob):
        for j in range(CH_ROWS):
            pltpu.async_copy(o_v.at[ob, j], acc_sh.at[r_v.at[ib, j]],
                             sc_sems.at[ob], add=True)

    def wait_scatter(ib, ob):
        for j in range(CH_ROWS):
            pltpu.make_async_copy(o_v.at[ob, j], acc_sh.at[r_v.at[ib, j]],
                                  sc_sems.at[ob]).wait()

    issue_inputs(0, 0)
    issue_inputs(1, 1)

    off = sid * SEG
    for po, pn in PIECES:
        pltpu.sync_copy(z_hbm.at[pl.ds(off + po, pn)],
                        nbuf_v.at[pl.ds(0, pn)])

        def _pk(i, c):
            zq = nbuf_v[pl.ds(i * LANES, LANES)]
            zero_v[pl.ds(i * LANES, LANES)] = plsc.bitcast(
                plsc.load_gather(ptab_v, [zq]), jnp.float32)
            return c
        lax.fori_loop(0, pn // LANES, _pk, 0, unroll=2)
        pltpu.sync_copy(zero_v.at[pl.ds(0, pn)],
                        out_hbm.at[pl.ds(cid * ACC_N + off + po, pn)])

    def _zero(i, c):
        zero_v[pl.ds(i * LANES, LANES)] = jnp.zeros((LANES,), jnp.float32)
        return c
    lax.fori_loop(0, PIECE // LANES, _zero, 0)
    for po, pn in PIECES:
        pltpu.sync_copy(zero_v.at[pl.ds(0, pn)],
                        acc_sh.at[pl.ds(off + po, pn)])
    plsc.subcore_barrier()
    pltpu.sync_copy(out_hbm.at[pl.ds(cid * ACC_N, ACC_N)], pk_v)

    s_x = par_v[0]
    c0, c1, c2, c3 = par_v[1], par_v[2], par_v[3], par_v[4]
    nd0, nd1, nd2, nd3 = par_v[5], par_v[6], par_v[7], par_v[8]
    inv_rsc = par_v[9]

    def compute_chunk(ib, ob):
        sref, rref, dref, oref = s_v.at[ib], r_v.at[ib], d_v.at[ib], o_v.at[ob]

        def _vec(i, c):
            row = i // (ROW // LANES)
            col = (i % (ROW // LANES)) * LANES
            s = sref[row, pl.ds(col, LANES)]
            r = rref[row, pl.ds(col, LANES)]
            dd = dref[row, pl.ds(col, LANES)]
            pku = plsc.bitcast(plsc.load_gather(pk_v, [s]), jnp.int32)
            pkv = plsc.bitcast(plsc.load_gather(pk_v, [r]), jnp.int32)
            zz = ((pku & 127) * (pkv & 127)).astype(jnp.float32)
            rq = ((lax.shift_right_logical(pku, 7) & 0x1FFF)
                  + (lax.shift_right_logical(pkv, 7) & 0x1FFF))
            pq = (lax.shift_right_logical(pku, 20)
                  + lax.shift_right_logical(pkv, 20))
            x = dd * pq.astype(jnp.float32) * s_x
            phi = (c0 * jnp.exp(nd0 * x) + c1 * jnp.exp(nd1 * x)
                   + c2 * jnp.exp(nd2 * x) + c3 * jnp.exp(nd3 * x))
            y = dd * inv_rsc / rq.astype(jnp.float32)
            y2 = y * y
            y4 = y2 * y2
            y6 = y4 * y2
            env = 1.0 - 28.0 * y6 + 48.0 * y6 * y - 21.0 * y4 * y4
            env = jnp.where(y < 1.0, env, 0.0)
            oref[row, pl.ds(col, LANES)] = 7.1998 * zz * phi * env / dd
            return c
        lax.fori_loop(0, CH // LANES, _vec, 0, unroll=2)

    def _group(g, carry):
        for b in range(PERIOD):
            c = g * PERIOD + b
            ib, ob = b % NBUF, b % OBUF
            wait_inputs(ib)
            compute_chunk(ib, ob)
            issue_scatter(ib, ob)
            @pl.when(c >= 2)
            def _():
                wait_scatter((b - 2) % NBUF, (b - 2) % OBUF)

            issue_inputs(c + 2, (b + 2) % NBUF)
        return carry

    lax.fori_loop(0, CHUNKS // PERIOD, _group, 0)
    for c in range((CHUNKS // PERIOD) * PERIOD, CHUNKS):
        ib, ob = c % NBUF, c % OBUF
        wait_inputs(ib)
        compute_chunk(ib, ob)
        issue_scatter(ib, ob)
        wait_scatter((c - 2) % NBUF, (c - 2) % OBUF)
        if c + 2 < CHUNKS:
            issue_inputs(c + 2, (c + 2) % NBUF)
    wait_scatter((CHUNKS - 2) % NBUF, (CHUNKS - 2) % OBUF)
    wait_scatter((CHUNKS - 1) % NBUF, (CHUNKS - 1) % OBUF)

    @pl.when(wid < N_EXTRA)
    def _():
        rbx = EXTRA_ROW0 + wid * CH_ROWS
        pltpu.sync_copy(send_hbm.at[pl.ds(rbx, CH_ROWS)], s_v.at[0])
        pltpu.sync_copy(recv_hbm.at[pl.ds(rbx, CH_ROWS)], r_v.at[0])
        pltpu.sync_copy(dist_hbm.at[pl.ds(rbx, CH_ROWS)], d_v.at[0])
        compute_chunk(0, 0)
        issue_scatter(0, 0)
        wait_scatter(0, 0)

    plsc.subcore_barrier()
    for po, pn in PIECES:
        pltpu.sync_copy(acc_sh.at[pl.ds(off + po, pn)],
                        zero_v.at[pl.ds(0, pn)])
        pltpu.sync_copy(zero_v.at[pl.ds(0, pn)],
                        out_hbm.at[pl.ds(cid * ACC_N + off + po, pn)])


def kernel(z, edge_distance, edge_index, a_factor, Z_power, screen_coefs,
           screen_exps, covalent_radii):
    idx = jnp.arange(TAB, dtype=jnp.float32)
    zpow = idx ** Z_power
    rad = jnp.pad(covalent_radii.astype(jnp.float32),
                  (0, TAB - covalent_radii.shape[0]))
    p_scale = jnp.max(zpow) / 4095.0
    r_scale = jnp.max(rad) / 8191.0
    pq = jnp.round(zpow / p_scale).astype(jnp.uint32)
    rq = jnp.round(rad / r_scale).astype(jnp.uint32)
    ptab = (pq << 20) | (rq << 7) | jnp.arange(TAB, dtype=jnp.uint32)
    ptab = lax.bitcast_convert_type(ptab, jnp.int32)

    inv_ab = 1.0 / (0.529 * a_factor.astype(jnp.float32))
    par = jnp.concatenate([
        (p_scale * inv_ab)[None],
        screen_coefs.astype(jnp.float32),
        -screen_exps.astype(jnp.float32),
        (1.0 / r_scale)[None],
    ])
    par2 = jnp.broadcast_to(par[:, None], (10, LANES)).astype(jnp.float32)

    z_pad = jnp.pad(z.astype(jnp.int32), (0, ACC_N - N_NODES))
    send = edge_index[0].reshape(ROWS, ROW)
    recv = edge_index[1].reshape(ROWS, ROW)
    dist = edge_distance.astype(jnp.float32).reshape(ROWS, ROW)

    mesh = plsc.VectorSubcoreMesh(core_axis_name="c", subcore_axis_name="s",
                                  num_cores=NC, num_subcores=NS)
    run = pl.kernel(
        _zbl_body,
        out_type=jax.ShapeDtypeStruct((NC * ACC_N,), jnp.float32),
        mesh=mesh,
        compiler_params=pltpu.CompilerParams(needs_layout_passes=False),
        scratch_types=[
            pltpu.VMEM((ACC_N,), jnp.float32),
            pltpu.VMEM((TAB,), jnp.int32),
            pltpu.VMEM((10, LANES), jnp.float32),
            pltpu.VMEM((NBUF, CH_ROWS, ROW), jnp.int32),
            pltpu.VMEM((NBUF, CH_ROWS, ROW), jnp.int32),
            pltpu.VMEM((NBUF, CH_ROWS, ROW), jnp.float32),
            pltpu.VMEM((OBUF, CH_ROWS, ROW), jnp.float32),
            pltpu.VMEM((PIECE,), jnp.float32),
            pltpu.VMEM((PIECE,), jnp.int32),
            pltpu.VMEM_SHARED((ACC_N,), jnp.float32),
            pltpu.SemaphoreType.DMA((NBUF,)),
            pltpu.SemaphoreType.DMA((OBUF,)),
        ],
    )
    partial = run(z_pad, send, recv, dist, ptab, par2)
    return partial[:N_NODES] + partial[ACC_N:ACC_N + N_NODES]


# speedup vs baseline: 1.1115x; 1.0076x over previous
import jax
import jax.numpy as jnp
from jax import lax
from jax.experimental import pallas as pl
from jax.experimental.pallas import tpu as pltpu
from jax.experimental.pallas import tpu_sc as plsc

N_NODES = 100000
N_EDGES = 6400000
NC, NS, LANES = 2, 16, 16
NW = NC * NS
ROW = 128
CH_ROWS = 8
CH = CH_ROWS * ROW
ROWS = N_EDGES // ROW
CHUNKS = 195
TILE_ROWS = CHUNKS * CH_ROWS
EXTRA_ROW0 = NW * TILE_ROWS
N_EXTRA = (ROWS - EXTRA_ROW0) // CH_ROWS
SEG = 6256
ACC_N = SEG * NS
TAB = 128
NBUF = 3
OBUF = 2
PERIOD = 6
PIECE = 2048
PIECES = ((0, 2048), (2048, 2048), (4096, 2048), (6144, 112))


def _zbl_body(z_hbm, send_hbm, recv_hbm, dist_hbm, ptab_hbm, par_hbm,
              out_hbm,
              pk_v, ptab_v, par_v, s_v, r_v, d_v, o_v, zero_v, nbuf_v,
              acc_sh, in_sems, sc_sems):
    cid = lax.axis_index("c")
    sid = lax.axis_index("s")
    wid = sid * NC + cid
    row0 = wid * TILE_ROWS

    pltpu.sync_copy(ptab_hbm, ptab_v)
    pltpu.sync_copy(par_hbm, par_v)

    def issue_inputs(c, b):
        rb = row0 + c * CH_ROWS
        pltpu.async_copy(send_hbm.at[pl.ds(rb, CH_ROWS)], s_v.at[b],
                         in_sems.at[b])
        pltpu.async_copy(recv_hbm.at[pl.ds(rb, CH_ROWS)], r_v.at[b],
                         in_sems.at[b])
        pltpu.async_copy(dist_hbm.at[pl.ds(rb, CH_ROWS)], d_v.at[b],
                         in_sems.at[b])

    def wait_inputs(b):
        pltpu.make_async_copy(send_hbm.at[pl.ds(0, CH_ROWS)], s_v.at[b],
                              in_sems.at[b]).wait()
        pltpu.make_async_copy(recv_hbm.at[pl.ds(0, CH_ROWS)], r_v.at[b],
                              in_sems.at[b]).wait()
        pltpu.make_async_copy(dist_hbm.at[pl.ds(0, CH_ROWS)], d_v.at[b],
                              in_sems.at[b]).wait()

    def issue_scatter(ib, ob):
        for j in range(CH_ROWS):
            pltpu.async_copy(o_v.at[ob, j], acc_sh.at[r_v.at[ib, j]],
                             sc_sems.at[ob], add=True)

    def wait_scatter(ib, ob):
        for j in range(CH_ROWS):
            pltpu.make_async_copy(o_v.at[ob, j], acc_sh.at[r_v.at[ib, j]],
                                  sc_sems.at[ob]).wait()

    issue_inputs(0, 0)
    issue_inputs(1, 1)

    off = sid * SEG
    for po, pn in PIECES:
        pltpu.sync_copy(z_hbm.at[pl.ds(off + po, pn)],
                        nbuf_v.at[pl.ds(0, pn)])

        def _pk(i, c):
            zq = nbuf_v[pl.ds(i * LANES, LANES)]
            zero_v[pl.ds(i * LANES, LANES)] = plsc.bitcast(
                plsc.load_gather(ptab_v, [zq]), jnp.float32)
            return c
        lax.fori_loop(0, pn // LANES, _pk, 0, unroll=2)
        pltpu.sync_copy(zero_v.at[pl.ds(0, pn)],
                        out_hbm.at[pl.ds(cid * ACC_N + off + po, pn)])

    def _zero(i, c):
        zero_v[pl.ds(i * LANES, LANES)] = jnp.zeros((LANES,), jnp.float32)
        return c
    lax.fori_loop(0, PIECE // LANES, _zero, 0)
    for po, pn in PIECES:
        pltpu.sync_copy(zero_v.at[pl.ds(0, pn)],
                        acc_sh.at[pl.ds(off + po, pn)])
    plsc.subcore_barrier()
    pltpu.sync_copy(out_hbm.at[pl.ds(cid * ACC_N, ACC_N)], pk_v)

    s_x = par_v[0]
    c0, c1, c2, c3 = par_v[1], par_v[2], par_v[3], par_v[4]
    nd0, nd1, nd2, nd3 = par_v[5], par_v[6], par_v[7], par_v[8]
    inv_rsc = par_v[9]

    def compute_chunk(ib, ob):
        sref, rref, dref, oref = s_v.at[ib], r_v.at[ib], d_v.at[ib], o_v.at[ob]

        def _vec(i, c):
            row = i // (ROW // LANES)
            col = (i % (ROW // LANES)) * LANES
            s = sref[row, pl.ds(col, LANES)]
            r = rref[row, pl.ds(col, LANES)]
            dd = dref[row, pl.ds(col, LANES)]
            pku = plsc.bitcast(plsc.load_gather(pk_v, [s]), jnp.int32)
            pkv = plsc.bitcast(plsc.load_gather(pk_v, [r]), jnp.int32)
            zz = ((pku & 127) * (pkv & 127)).astype(jnp.float32)
            rq = ((lax.shift_right_logical(pku, 7) & 0x1FFF)
                  + (lax.shift_right_logical(pkv, 7) & 0x1FFF))
            pq = (lax.shift_right_logical(pku, 20)
                  + lax.shift_right_logical(pkv, 20))
            x = dd * pq.astype(jnp.float32) * s_x
            phi = (c0 * jnp.exp(nd0 * x) + c1 * jnp.exp(nd1 * x)
                   + c2 * jnp.exp(nd2 * x) + c3 * jnp.exp(nd3 * x))
            y = dd * inv_rsc / rq.astype(jnp.float32)
            y2 = y * y
            y4 = y2 * y2
            y6 = y4 * y2
            env = 1.0 - 28.0 * y6 + 48.0 * y6 * y - 21.0 * y4 * y4
            env = jnp.where(y < 1.0, env, 0.0)
            oref[row, pl.ds(col, LANES)] = 7.1998 * zz * phi * env / dd
            return c
        lax.fori_loop(0, CH // LANES, _vec, 0, unroll=2)

    def _group(g, carry):
        for b in range(PERIOD):
            c = g * PERIOD + b
            ib, ob = b % NBUF, b % OBUF
            wait_inputs(ib)
            compute_chunk(ib, ob)
            issue_scatter(ib, ob)
            @pl.when(c >= 1)
            def _():
                wait_scatter((b - 1) % NBUF, (b - 1) % OBUF)

            issue_inputs(c + 2, (b + 2) % NBUF)
        return carry

    lax.fori_loop(0, CHUNKS // PERIOD, _group, 0)
    for c in range((CHUNKS // PERIOD) * PERIOD, CHUNKS):
        ib, ob = c % NBUF, c % OBUF
        wait_inputs(ib)
        compute_chunk(ib, ob)
        issue_scatter(ib, ob)
        wait_scatter((c - 1) % NBUF, (c - 1) % OBUF)
        if c + 2 < CHUNKS:
            issue_inputs(c + 2, (c + 2) % NBUF)
    wait_scatter((CHUNKS - 1) % NBUF, (CHUNKS - 1) % OBUF)

    @pl.when(wid < N_EXTRA)
    def _():
        rbx = EXTRA_ROW0 + wid * CH_ROWS
        pltpu.sync_copy(send_hbm.at[pl.ds(rbx, CH_ROWS)], s_v.at[0])
        pltpu.sync_copy(recv_hbm.at[pl.ds(rbx, CH_ROWS)], r_v.at[0])
        pltpu.sync_copy(dist_hbm.at[pl.ds(rbx, CH_ROWS)], d_v.at[0])
        compute_chunk(0, 0)
        issue_scatter(0, 0)
        wait_scatter(0, 0)

    plsc.subcore_barrier()
    for po, pn in PIECES:
        pltpu.sync_copy(acc_sh.at[pl.ds(off + po, pn)],
                        zero_v.at[pl.ds(0, pn)])
        pltpu.sync_copy(zero_v.at[pl.ds(0, pn)],
                        out_hbm.at[pl.ds(cid * ACC_N + off + po, pn)])


def kernel(z, edge_distance, edge_index, a_factor, Z_power, screen_coefs,
           screen_exps, covalent_radii):
    idx = jnp.arange(TAB, dtype=jnp.float32)
    zpow = idx ** Z_power
    rad = jnp.pad(covalent_radii.astype(jnp.float32),
                  (0, TAB - covalent_radii.shape[0]))
    p_scale = jnp.max(zpow) / 4095.0
    r_scale = jnp.max(rad) / 8191.0
    pq = jnp.round(zpow / p_scale).astype(jnp.uint32)
    rq = jnp.round(rad / r_scale).astype(jnp.uint32)
    ptab = (pq << 20) | (rq << 7) | jnp.arange(TAB, dtype=jnp.uint32)
    ptab = lax.bitcast_convert_type(ptab, jnp.int32)

    inv_ab = 1.0 / (0.529 * a_factor.astype(jnp.float32))
    par = jnp.concatenate([
        (p_scale * inv_ab)[None],
        screen_coefs.astype(jnp.float32),
        -screen_exps.astype(jnp.float32),
        (1.0 / r_scale)[None],
    ])
    par2 = jnp.broadcast_to(par[:, None], (10, LANES)).astype(jnp.float32)

    z_pad = jnp.pad(z.astype(jnp.int32), (0, ACC_N - N_NODES))
    send = edge_index[0].reshape(ROWS, ROW)
    recv = edge_index[1].reshape(ROWS, ROW)
    dist = edge_distance.astype(jnp.float32).reshape(ROWS, ROW)

    mesh = plsc.VectorSubcoreMesh(core_axis_name="c", subcore_axis_name="s",
                                  num_cores=NC, num_subcores=NS)
    run = pl.kernel(
        _zbl_body,
        out_type=jax.ShapeDtypeStruct((NC * ACC_N,), jnp.float32),
        mesh=mesh,
        compiler_params=pltpu.CompilerParams(needs_layout_passes=False),
        scratch_types=[
            pltpu.VMEM((ACC_N,), jnp.float32),
            pltpu.VMEM((TAB,), jnp.int32),
            pltpu.VMEM((10, LANES), jnp.float32),
            pltpu.VMEM((NBUF, CH_ROWS, ROW), jnp.int32),
            pltpu.VMEM((NBUF, CH_ROWS, ROW), jnp.int32),
            pltpu.VMEM((NBUF, CH_ROWS, ROW), jnp.float32),
            pltpu.VMEM((OBUF, CH_ROWS, ROW), jnp.float32),
            pltpu.VMEM((PIECE,), jnp.float32),
            pltpu.VMEM((PIECE,), jnp.int32),
            pltpu.VMEM_SHARED((ACC_N,), jnp.float32),
            pltpu.SemaphoreType.DMA((NBUF,)),
            pltpu.SemaphoreType.DMA((OBUF,)),
        ],
    )
    partial = run(z_pad, send, recv, dist, ptab, par2)
    return partial[:N_NODES] + partial[ACC_N:ACC_N + N_NODES]
